# Initial kernel scaffold; baseline (speedup 1.0000x reference)
import jax
import jax.numpy as jnp
from jax.experimental import pallas as pl


def kernel(x, edge_index, W1, b1, W2, b2, W3, b3):
    raise NotImplementedError("write your pallas kernel here")


# speedup vs baseline: 6.3775x; 6.3775x over previous
import functools

import jax
import jax.numpy as jnp
from jax import lax
from jax.experimental import pallas as pl
from jax.experimental.pallas import tpu as pltpu
from jax.experimental.pallas import tpu_sc as plsc

N = 10000
F_IN = 128
HID = 256
NC = 2
NS = 16
NW = NC * NS
CK = 128
GP = 8
N_PAD = 10112
ROWS_T = N_PAD // NS

_mesh = functools.partial(
    plsc.VectorSubcoreMesh,
    core_axis_name="c", subcore_axis_name="s", num_cores=NC, num_subcores=NS,
)


def _wid():
    return lax.axis_index("c") * NS + lax.axis_index("s")


def _make_sc_deg(groups):
    @functools.partial(
        pl.kernel,
        out_type=jax.ShapeDtypeStruct((NC, N_PAD, F_IN), jnp.float32),
        mesh=_mesh(),
        scratch_types=[
            pltpu.VMEM((groups, GP, CK), jnp.int32),
            pltpu.VMEM((CK, F_IN), jnp.float32),
            pltpu.VMEM_SHARED((N_PAD, F_IN), jnp.float32),
        ],
    )
    def sc_deg(dst_hbm, ones_hbm, zeros_hbm, out_hbm, dst_v, ones_v, acc):
        c = lax.axis_index("c")
        s = lax.axis_index("s")
        w = _wid()
        pltpu.sync_copy(zeros_hbm.at[pl.ds(s * ROWS_T, ROWS_T)],
                        acc.at[pl.ds(s * ROWS_T, ROWS_T)])
        pltpu.sync_copy(dst_hbm.at[w], dst_v)
        pltpu.sync_copy(ones_hbm, ones_v)
        plsc.subcore_barrier()

        def body(g, carry):
            for j in range(GP):
                pltpu.sync_copy(ones_v, acc.at[dst_v.at[g, j]], add=True)
            return carry

        lax.fori_loop(0, groups, body, 0)
        plsc.subcore_barrier()
        pltpu.sync_copy(acc.at[pl.ds(s * ROWS_T, ROWS_T)],
                        out_hbm.at[c, pl.ds(s * ROWS_T, ROWS_T)])

    return sc_deg


def _make_sc_agg(groups):
    @functools.partial(
        pl.kernel,
        out_type=jax.ShapeDtypeStruct((NC, N_PAD, F_IN), jnp.float32),
        mesh=_mesh(),
        scratch_types=[
            pltpu.VMEM((2, GP, CK), jnp.int32),
            pltpu.VMEM((2, GP, CK), jnp.int32),
            pltpu.VMEM((2, CK, F_IN), jnp.float32),
            pltpu.VMEM_SHARED((N_PAD, F_IN), jnp.float32),
            pltpu.SemaphoreType.DMA,
            pltpu.SemaphoreType.DMA,
            pltpu.SemaphoreType.DMA,
            pltpu.SemaphoreType.DMA,
        ],
    )
    def sc_agg(table_hbm, src_hbm, dst_hbm, zeros_hbm, out_hbm,
               src_v, dst_v, rows, acc, sem0, sem1, semis, semid):
        c = lax.axis_index("c")
        s = lax.axis_index("s")
        w = _wid()
        pltpu.sync_copy(zeros_hbm.at[pl.ds(s * ROWS_T, ROWS_T)],
                        acc.at[pl.ds(s * ROWS_T, ROWS_T)])
        pltpu.sync_copy(src_hbm.at[w, 0], src_v.at[0])
        pltpu.sync_copy(dst_hbm.at[w, 0], dst_v.at[0])
        plsc.subcore_barrier()

        sems = (sem0, sem1)

        def gather(buf, idx, sem):
            return pltpu.async_copy(table_hbm.at[idx], rows.at[buf], sem)

        def gwait(buf, idx, sem):
            pltpu.make_async_copy(table_hbm.at[idx], rows.at[buf], sem).wait()

        gather(0, src_v.at[0, 0], sem0)

        def body(g, carry):
            cur = g % 2
            nxt = (g + 1) % 2

            @pl.when(g < groups - 1)
            def _prefetch():
                pltpu.async_copy(src_hbm.at[w, g + 1], src_v.at[nxt], semis)
                pltpu.async_copy(dst_hbm.at[w, g + 1], dst_v.at[nxt], semid)

            for j in range(GP):
                p = j % 2
                if j < GP - 1:
                    gather((j + 1) % 2, src_v.at[cur, j + 1], sems[(j + 1) % 2])
                else:
                    @pl.when(g < groups - 1)
                    def _next_group():
                        pltpu.make_async_copy(src_hbm.at[w, g + 1],
                                              src_v.at[nxt], semis).wait()
                        pltpu.make_async_copy(dst_hbm.at[w, g + 1],
                                              dst_v.at[nxt], semid).wait()
                        gather(0, src_v.at[nxt, 0], sem0)
                gwait(p, src_v.at[cur, j], sems[p])
                pltpu.sync_copy(rows.at[p], acc.at[dst_v.at[cur, j]], add=True)
            return carry

        lax.fori_loop(0, groups, body, 0)
        plsc.subcore_barrier()
        pltpu.sync_copy(acc.at[pl.ds(s * ROWS_T, ROWS_T)],
                        out_hbm.at[c, pl.ds(s * ROWS_T, ROWS_T)])

    return sc_agg


_BLK = 1264
_GRID = N_PAD // _BLK


def _row_spec(w):
    return pl.BlockSpec((_BLK, w), lambda i: (i, 0))


def _part_spec(w):
    return pl.BlockSpec((NC, _BLK, w), lambda i: (0, i, 0))


def _full_spec(h, w):
    return pl.BlockSpec((h, w), lambda i: (0, 0))


def _tc_prep_body(x_ref, degp_ref, dinv_ref, xs_ref):
    deg = 1.0 + degp_ref[0, :, 0:1] + degp_ref[1, :, 0:1]
    dinv = lax.rsqrt(deg)
    dinv_ref[...] = dinv
    xs_ref[...] = x_ref[...] * dinv


def _tc_prep(x, degp):
    return pl.pallas_call(
        _tc_prep_body,
        grid=(_GRID,),
        in_specs=[_row_spec(F_IN), _part_spec(F_IN)],
        out_specs=[_row_spec(1), _row_spec(F_IN)],
        out_shape=[
            jax.ShapeDtypeStruct((N_PAD, 1), jnp.float32),
            jax.ShapeDtypeStruct((N_PAD, F_IN), jnp.float32),
        ],
    )(x, degp)


def _tc_l1_body(y1_ref, xs1_ref, dinv_ref, w1_ref, b1_ref, w2_ref,
                a_ref, b_ref):
    dinv = dinv_ref[...]
    z1 = dinv * (y1_ref[0] + y1_ref[1] + xs1_ref[...])
    h1 = jnp.maximum(
        jnp.dot(z1, w1_ref[...], preferred_element_type=jnp.float32)
        + b1_ref[...], 0.0)
    g2 = jnp.dot(h1, w2_ref[...], preferred_element_type=jnp.float32)
    xs2 = dinv * g2
    a_ref[...] = xs2[:, :F_IN]
    b_ref[...] = xs2[:, F_IN:]


def _tc_l1(y1, xs1, dinv, W1, b1, W2):
    return pl.pallas_call(
        _tc_l1_body,
        grid=(_GRID,),
        in_specs=[_part_spec(F_IN), _row_spec(F_IN), _row_spec(1),
                  _full_spec(F_IN, HID), _full_spec(1, HID),
                  _full_spec(HID, HID)],
        out_specs=[_row_spec(F_IN), _row_spec(F_IN)],
        out_shape=[
            jax.ShapeDtypeStruct((N_PAD, F_IN), jnp.float32),
            jax.ShapeDtypeStruct((N_PAD, F_IN), jnp.float32),
        ],
    )(y1, xs1, dinv, W1, b1, W2)


def _tc_l2_body(y2a_ref, y2b_ref, xs2a_ref, xs2b_ref, dinv_ref, b2_ref,
                w3_ref, xs3_ref):
    dinv = dinv_ref[...]
    z2a = dinv * (y2a_ref[0] + y2a_ref[1] + xs2a_ref[...]) + b2_ref[:, :F_IN]
    z2b = dinv * (y2b_ref[0] + y2b_ref[1] + xs2b_ref[...]) + b2_ref[:, F_IN:]
    h2 = jnp.maximum(jnp.concatenate([z2a, z2b], axis=1), 0.0)
    g3 = jnp.dot(h2, w3_ref[...], preferred_element_type=jnp.float32)
    xs3_ref[...] = dinv * g3


def _tc_l2(y2a, y2b, xs2a, xs2b, dinv, b2, W3):
    return pl.pallas_call(
        _tc_l2_body,
        grid=(_GRID,),
        in_specs=[_part_spec(F_IN), _part_spec(F_IN), _row_spec(F_IN),
                  _row_spec(F_IN), _row_spec(1), _full_spec(1, HID),
                  _full_spec(HID, F_IN)],
        out_specs=[_row_spec(F_IN)],
        out_shape=[jax.ShapeDtypeStruct((N_PAD, F_IN), jnp.float32)],
    )(y2a, y2b, xs2a, xs2b, dinv, b2, W3)[0]


def _tc_fin_body(y3_ref, xs3_ref, dinv_ref, b3_ref, out_ref):
    dinv = dinv_ref[...]
    out_ref[...] = dinv * (y3_ref[0] + y3_ref[1] + xs3_ref[...]) + b3_ref[...]


def _tc_fin(y3, xs3, dinv, b3):
    return pl.pallas_call(
        _tc_fin_body,
        grid=(_GRID,),
        in_specs=[_part_spec(F_IN), _row_spec(F_IN), _row_spec(1),
                  _full_spec(1, F_IN)],
        out_specs=[_row_spec(F_IN)],
        out_shape=[jax.ShapeDtypeStruct((N_PAD, F_IN), jnp.float32)],
    )(y3, xs3, dinv, b3)[0]


def kernel(x, edge_index, W1, b1, W2, b2, W3, b3):
    E = edge_index.shape[1]
    per_w = CK * NW
    groups = -(-E // (per_w * GP))
    ep = groups * GP * per_w

    src = edge_index[0]
    dst = edge_index[1]
    pad = ep - E
    src_slab = jnp.concatenate(
        [src, jnp.zeros((pad,), jnp.int32)]).reshape(NW, groups, GP, CK)
    dst_slab = jnp.concatenate(
        [dst, jnp.full((pad,), N, jnp.int32)]).reshape(NW, groups, GP, CK)

    zeros128 = jnp.zeros((N_PAD, F_IN), jnp.float32)
    ones128 = jnp.ones((CK, F_IN), jnp.float32)
    xp = jnp.concatenate([x, jnp.zeros((N_PAD - N, F_IN), jnp.float32)])

    sc_deg = _make_sc_deg(groups)
    sc_agg = _make_sc_agg(groups)

    degp = sc_deg(dst_slab, ones128, zeros128)
    dinv, xs1 = _tc_prep(xp, degp)

    y1 = sc_agg(xs1, src_slab, dst_slab, zeros128)
    xs2a, xs2b = _tc_l1(y1, xs1, dinv, W1, b1.reshape(1, HID), W2)

    y2a = sc_agg(xs2a, src_slab, dst_slab, zeros128)
    y2b = sc_agg(xs2b, src_slab, dst_slab, zeros128)
    xs3 = _tc_l2(y2a, y2b, xs2a, xs2b, dinv, b2.reshape(1, HID), W3)

    y3 = sc_agg(xs3, src_slab, dst_slab, zeros128)
    return _tc_fin(y3, xs3, dinv, b3.reshape(1, F_IN))[:N]


# speedup vs baseline: 8.0327x; 1.2595x over previous
import functools

import jax
import jax.numpy as jnp
from jax import lax
from jax.experimental import pallas as pl
from jax.experimental.pallas import tpu as pltpu
from jax.experimental.pallas import tpu_sc as plsc

N = 10000
F_IN = 128
HID = 256
NC = 2
NS = 16
NW = NC * NS
CK = 128
GP = 8
N_PAD = 10112
ROWS_T = N_PAD // NS

_mesh = functools.partial(
    plsc.VectorSubcoreMesh,
    core_axis_name="c", subcore_axis_name="s", num_cores=NC, num_subcores=NS,
)


def _wid():
    return lax.axis_index("c") * NS + lax.axis_index("s")


def _make_sc_deg(groups):
    @functools.partial(
        pl.kernel,
        out_type=jax.ShapeDtypeStruct((NC, N_PAD, F_IN), jnp.float32),
        mesh=_mesh(),
        scratch_types=[
            pltpu.VMEM((groups, GP, CK), jnp.int32),
            pltpu.VMEM((CK, F_IN), jnp.float32),
            pltpu.VMEM_SHARED((N_PAD, F_IN), jnp.float32),
        ],
    )
    def sc_deg(dst_hbm, ones_hbm, zeros_hbm, out_hbm, dst_v, ones_v, acc):
        c = lax.axis_index("c")
        s = lax.axis_index("s")
        w = _wid()
        pltpu.sync_copy(zeros_hbm.at[pl.ds(s * ROWS_T, ROWS_T)],
                        acc.at[pl.ds(s * ROWS_T, ROWS_T)])
        pltpu.sync_copy(dst_hbm.at[w], dst_v)
        pltpu.sync_copy(ones_hbm, ones_v)
        plsc.subcore_barrier()

        def body(g, carry):
            for j in range(GP):
                pltpu.sync_copy(ones_v, acc.at[dst_v.at[g, j]], add=True)
            return carry

        lax.fori_loop(0, groups, body, 0)
        plsc.subcore_barrier()
        pltpu.sync_copy(acc.at[pl.ds(s * ROWS_T, ROWS_T)],
                        out_hbm.at[c, pl.ds(s * ROWS_T, ROWS_T)])

    return sc_deg


def _make_sc_agg(groups, ntab):
    @functools.partial(
        pl.kernel,
        out_type=jax.ShapeDtypeStruct((ntab, NC, N_PAD, F_IN), jnp.float32),
        mesh=_mesh(),
        scratch_types=[
            pltpu.VMEM((2, GP, CK), jnp.int32),
            pltpu.VMEM((2, GP, CK), jnp.int32),
            pltpu.VMEM((2, CK, F_IN), jnp.float32),
            pltpu.VMEM_SHARED((N_PAD, F_IN), jnp.float32),
            pltpu.SemaphoreType.DMA,
            pltpu.SemaphoreType.DMA,
            pltpu.SemaphoreType.DMA,
            pltpu.SemaphoreType.DMA,
        ],
    )
    def sc_agg(*refs):
        tables = refs[:ntab]
        src_hbm, dst_hbm, zeros_hbm, out_hbm = refs[ntab:ntab + 4]
        (src_v, dst_v, rows, acc, sem0, sem1, semis, semid) = refs[ntab + 4:]
        c = lax.axis_index("c")
        s = lax.axis_index("s")
        w = _wid()
        gsems = (sem0, sem1)

        for t in range(ntab):
            table_hbm = tables[t]
            pltpu.sync_copy(zeros_hbm.at[pl.ds(s * ROWS_T, ROWS_T)],
                            acc.at[pl.ds(s * ROWS_T, ROWS_T)])
            pltpu.sync_copy(src_hbm.at[w, 0], src_v.at[0])
            pltpu.sync_copy(dst_hbm.at[w, 0], dst_v.at[0])
            plsc.subcore_barrier()

            def gather(buf, idx, sem):
                return pltpu.async_copy(table_hbm.at[idx], rows.at[buf], sem)

            def gwait(buf, idx, sem):
                pltpu.make_async_copy(table_hbm.at[idx], rows.at[buf],
                                      sem).wait()

            gather(0, src_v.at[0, 0], sem0)

            def body(g, carry):
                cur = g % 2
                nxt = (g + 1) % 2

                @pl.when(g < groups - 1)
                def _prefetch():
                    pltpu.async_copy(src_hbm.at[w, g + 1], src_v.at[nxt],
                                     semis)
                    pltpu.async_copy(dst_hbm.at[w, g + 1], dst_v.at[nxt],
                                     semid)

                for j in range(GP):
                    p = j % 2
                    if j < GP - 1:
                        gather(1 - p, src_v.at[cur, j + 1], gsems[1 - p])
                    else:
                        @pl.when(g < groups - 1)
                        def _next_group():
                            pltpu.make_async_copy(src_hbm.at[w, g + 1],
                                                  src_v.at[nxt], semis).wait()
                            pltpu.make_async_copy(dst_hbm.at[w, g + 1],
                                                  dst_v.at[nxt], semid).wait()
                            gather(0, src_v.at[nxt, 0], gsems[0])
                    gwait(p, src_v.at[cur, j], gsems[p])
                    pltpu.sync_copy(rows.at[p], acc.at[dst_v.at[cur, j]],
                                    add=True)
                return carry

            lax.fori_loop(0, groups, body, 0)
            plsc.subcore_barrier()
            pltpu.sync_copy(acc.at[pl.ds(s * ROWS_T, ROWS_T)],
                            out_hbm.at[t, c, pl.ds(s * ROWS_T, ROWS_T)])

    return sc_agg


_BLK = 1264
_GRID = N_PAD // _BLK


def _row_spec(w):
    return pl.BlockSpec((_BLK, w), lambda i: (i, 0))


def _part_spec(w):
    return pl.BlockSpec((NC, _BLK, w), lambda i: (0, i, 0))


def _part1_spec(w):
    return pl.BlockSpec((1, NC, _BLK, w), lambda i: (0, 0, i, 0))


def _full_spec(h, w):
    return pl.BlockSpec((h, w), lambda i: (0, 0))


def _tc_prep_body(x_ref, degp_ref, dinv_ref, xs_ref):
    deg = 1.0 + degp_ref[0, :, 0:1] + degp_ref[1, :, 0:1]
    dinv = lax.rsqrt(deg)
    dinv_ref[...] = dinv
    xs_ref[...] = x_ref[...] * dinv


def _tc_prep(x, degp):
    return pl.pallas_call(
        _tc_prep_body,
        grid=(_GRID,),
        in_specs=[_row_spec(F_IN), _part_spec(F_IN)],
        out_specs=[_row_spec(1), _row_spec(F_IN)],
        out_shape=[
            jax.ShapeDtypeStruct((N_PAD, 1), jnp.float32),
            jax.ShapeDtypeStruct((N_PAD, F_IN), jnp.float32),
        ],
    )(x, degp)


def _tc_l1_body(y1_ref, xs1_ref, dinv_ref, w1_ref, b1_ref, w2_ref,
                a_ref, b_ref):
    dinv = dinv_ref[...]
    z1 = dinv * (y1_ref[0, 0] + y1_ref[0, 1] + xs1_ref[...])
    h1 = jnp.maximum(
        jnp.dot(z1, w1_ref[...], preferred_element_type=jnp.float32)
        + b1_ref[...], 0.0)
    g2 = jnp.dot(h1, w2_ref[...], preferred_element_type=jnp.float32)
    xs2 = dinv * g2
    a_ref[...] = xs2[:, :F_IN]
    b_ref[...] = xs2[:, F_IN:]


def _tc_l1(y1, xs1, dinv, W1, b1, W2):
    return pl.pallas_call(
        _tc_l1_body,
        grid=(_GRID,),
        in_specs=[_part1_spec(F_IN), _row_spec(F_IN), _row_spec(1),
                  _full_spec(F_IN, HID), _full_spec(1, HID),
                  _full_spec(HID, HID)],
        out_specs=[_row_spec(F_IN), _row_spec(F_IN)],
        out_shape=[
            jax.ShapeDtypeStruct((N_PAD, F_IN), jnp.float32),
            jax.ShapeDtypeStruct((N_PAD, F_IN), jnp.float32),
        ],
    )(y1, xs1, dinv, W1, b1, W2)


def _tc_l2_body(y2_ref, xs2a_ref, xs2b_ref, dinv_ref, b2_ref,
                w3_ref, xs3_ref):
    dinv = dinv_ref[...]
    z2a = (dinv * (y2_ref[0, 0] + y2_ref[0, 1] + xs2a_ref[...])
           + b2_ref[:, :F_IN])
    z2b = (dinv * (y2_ref[1, 0] + y2_ref[1, 1] + xs2b_ref[...])
           + b2_ref[:, F_IN:])
    h2 = jnp.maximum(jnp.concatenate([z2a, z2b], axis=1), 0.0)
    g3 = jnp.dot(h2, w3_ref[...], preferred_element_type=jnp.float32)
    xs3_ref[...] = dinv * g3


def _tc_l2(y2, xs2a, xs2b, dinv, b2, W3):
    return pl.pallas_call(
        _tc_l2_body,
        grid=(_GRID,),
        in_specs=[pl.BlockSpec((2, NC, _BLK, F_IN), lambda i: (0, 0, i, 0)),
                  _row_spec(F_IN), _row_spec(F_IN), _row_spec(1),
                  _full_spec(1, HID), _full_spec(HID, F_IN)],
        out_specs=[_row_spec(F_IN)],
        out_shape=[jax.ShapeDtypeStruct((N_PAD, F_IN), jnp.float32)],
    )(y2, xs2a, xs2b, dinv, b2, W3)[0]


def _tc_fin_body(y3_ref, xs3_ref, dinv_ref, b3_ref, out_ref):
    dinv = dinv_ref[...]
    out_ref[...] = (dinv * (y3_ref[0, 0] + y3_ref[0, 1] + xs3_ref[...])
                    + b3_ref[...])


def _tc_fin(y3, xs3, dinv, b3):
    return pl.pallas_call(
        _tc_fin_body,
        grid=(_GRID,),
        in_specs=[_part1_spec(F_IN), _row_spec(F_IN), _row_spec(1),
                  _full_spec(1, F_IN)],
        out_specs=[_row_spec(F_IN)],
        out_shape=[jax.ShapeDtypeStruct((N_PAD, F_IN), jnp.float32)],
    )(y3, xs3, dinv, b3)[0]


def kernel(x, edge_index, W1, b1, W2, b2, W3, b3):
    E = edge_index.shape[1]
    per_w = CK * NW
    groups = -(-E // (per_w * GP))
    ep = groups * GP * per_w

    src = edge_index[0]
    dst = edge_index[1]
    pad = ep - E
    src_slab = jnp.concatenate(
        [src, jnp.zeros((pad,), jnp.int32)]).reshape(NW, groups, GP, CK)
    dst_slab = jnp.concatenate(
        [dst, jnp.full((pad,), N, jnp.int32)]).reshape(NW, groups, GP, CK)

    zeros128 = jnp.zeros((N_PAD, F_IN), jnp.float32)
    ones128 = jnp.ones((CK, F_IN), jnp.float32)
    xp = jnp.concatenate([x, jnp.zeros((N_PAD - N, F_IN), jnp.float32)])

    sc_deg = _make_sc_deg(groups)
    sc_agg1 = _make_sc_agg(groups, 1)
    sc_agg2 = _make_sc_agg(groups, 2)

    degp = sc_deg(dst_slab, ones128, zeros128)
    dinv, xs1 = _tc_prep(xp, degp)

    y1 = sc_agg1(xs1, src_slab, dst_slab, zeros128)
    xs2a, xs2b = _tc_l1(y1, xs1, dinv, W1, b1.reshape(1, HID), W2)

    y2 = sc_agg2(xs2a, xs2b, src_slab, dst_slab, zeros128)
    xs3 = _tc_l2(y2, xs2a, xs2b, dinv, b2.reshape(1, HID), W3)

    y3 = sc_agg1(xs3, src_slab, dst_slab, zeros128)
    return _tc_fin(y3, xs3, dinv, b3.reshape(1, F_IN))[:N]


# speedup vs baseline: 8.0355x; 1.0003x over previous
import functools

import jax
import jax.numpy as jnp
from jax import lax
from jax.experimental import pallas as pl
from jax.experimental.pallas import tpu as pltpu
from jax.experimental.pallas import tpu_sc as plsc

N = 10000
F_IN = 128
HID = 256
NC = 2
NS = 16
NW = NC * NS
CK = 128
GP = 8
N_PAD = 10112
ROWS_T = N_PAD // NS

_mesh = functools.partial(
    plsc.VectorSubcoreMesh,
    core_axis_name="c", subcore_axis_name="s", num_cores=NC, num_subcores=NS,
)


def _wid():
    return lax.axis_index("c") * NS + lax.axis_index("s")


def _make_sc_deg(groups):
    @functools.partial(
        pl.kernel,
        out_type=jax.ShapeDtypeStruct((NC, N_PAD, F_IN), jnp.float32),
        mesh=_mesh(),
        scratch_types=[
            pltpu.VMEM((groups, GP, CK), jnp.int32),
            pltpu.VMEM((CK, F_IN), jnp.float32),
            pltpu.VMEM_SHARED((N_PAD, F_IN), jnp.float32),
        ],
    )
    def sc_deg(dst_hbm, ones_hbm, zeros_hbm, out_hbm, dst_v, ones_v, acc):
        c = lax.axis_index("c")
        s = lax.axis_index("s")
        w = _wid()
        pltpu.sync_copy(zeros_hbm.at[pl.ds(s * ROWS_T, ROWS_T)],
                        acc.at[pl.ds(s * ROWS_T, ROWS_T)])
        pltpu.sync_copy(dst_hbm.at[w], dst_v)
        pltpu.sync_copy(ones_hbm, ones_v)
        plsc.subcore_barrier()

        def body(g, carry):
            for j in range(GP):
                pltpu.sync_copy(ones_v, acc.at[dst_v.at[g, j]], add=True)
            return carry

        lax.fori_loop(0, groups, body, 0)
        plsc.subcore_barrier()
        pltpu.sync_copy(acc.at[pl.ds(s * ROWS_T, ROWS_T)],
                        out_hbm.at[c, pl.ds(s * ROWS_T, ROWS_T)])

    return sc_deg


def _make_sc_agg(groups, ntab):
    @functools.partial(
        pl.kernel,
        out_type=jax.ShapeDtypeStruct((ntab, NC, N_PAD, F_IN), jnp.float32),
        mesh=_mesh(),
        scratch_types=[
            pltpu.VMEM((2, GP, CK), jnp.int32),
            pltpu.VMEM((2, GP, CK), jnp.int32),
            pltpu.VMEM((2, CK, F_IN), jnp.float32),
            pltpu.VMEM_SHARED((N_PAD, F_IN), jnp.float32),
            pltpu.SemaphoreType.DMA,
            pltpu.SemaphoreType.DMA,
            pltpu.SemaphoreType.DMA,
            pltpu.SemaphoreType.DMA,
        ],
    )
    def sc_agg(*refs):
        tables = refs[:ntab]
        src_hbm, dst_hbm, zeros_hbm, out_hbm = refs[ntab:ntab + 4]
        (src_v, dst_v, rows, acc, sem0, sem1, semis, semid) = refs[ntab + 4:]
        c = lax.axis_index("c")
        s = lax.axis_index("s")
        w = _wid()
        gsems = (sem0, sem1)

        for t in range(ntab):
            table_hbm = tables[t]
            pltpu.sync_copy(zeros_hbm.at[pl.ds(s * ROWS_T, ROWS_T)],
                            acc.at[pl.ds(s * ROWS_T, ROWS_T)])
            pltpu.sync_copy(src_hbm.at[w, 0], src_v.at[0])
            pltpu.sync_copy(dst_hbm.at[w, 0], dst_v.at[0])
            plsc.subcore_barrier()

            def gather(buf, idx, sem):
                return pltpu.async_copy(table_hbm.at[idx], rows.at[buf], sem)

            def gwait(buf, idx, sem):
                pltpu.make_async_copy(table_hbm.at[idx], rows.at[buf],
                                      sem).wait()

            gather(0, src_v.at[0, 0], sem0)

            def body(g, carry):
                cur = g % 2
                nxt = (g + 1) % 2

                @pl.when(g < groups - 1)
                def _prefetch():
                    pltpu.async_copy(src_hbm.at[w, g + 1], src_v.at[nxt],
                                     semis)
                    pltpu.async_copy(dst_hbm.at[w, g + 1], dst_v.at[nxt],
                                     semid)

                for j in range(GP):
                    p = j % 2
                    if j < GP - 1:
                        gather(1 - p, src_v.at[cur, j + 1], gsems[1 - p])
                    else:
                        @pl.when(g < groups - 1)
                        def _next_group():
                            pltpu.make_async_copy(src_hbm.at[w, g + 1],
                                                  src_v.at[nxt], semis).wait()
                            pltpu.make_async_copy(dst_hbm.at[w, g + 1],
                                                  dst_v.at[nxt], semid).wait()
                            gather(0, src_v.at[nxt, 0], gsems[0])
                    gwait(p, src_v.at[cur, j], gsems[p])
                    pltpu.sync_copy(rows.at[p], acc.at[dst_v.at[cur, j]],
                                    add=True)
                return carry

            lax.fori_loop(0, groups, body, 0)
            plsc.subcore_barrier()
            pltpu.sync_copy(acc.at[pl.ds(s * ROWS_T, ROWS_T)],
                            out_hbm.at[t, c, pl.ds(s * ROWS_T, ROWS_T)])

    return sc_agg


_BLK = 1264
_GRID = N_PAD // _BLK


def _row_spec(w):
    return pl.BlockSpec((_BLK, w), lambda i: (i, 0))


def _part_spec(w):
    return pl.BlockSpec((NC, _BLK, w), lambda i: (0, i, 0))


def _part1_spec(w):
    return pl.BlockSpec((1, NC, _BLK, w), lambda i: (0, 0, i, 0))


def _full_spec(h, w):
    return pl.BlockSpec((h, w), lambda i: (0, 0))


def _tc_prep_body(x_ref, degp_ref, dinv_ref, xs_ref):
    deg = 1.0 + degp_ref[0, :, 0:1] + degp_ref[1, :, 0:1]
    dinv = lax.rsqrt(deg)
    dinv_ref[...] = dinv
    xs_ref[...] = x_ref[...] * dinv


def _tc_prep(x, degp):
    return pl.pallas_call(
        _tc_prep_body,
        grid=(_GRID,),
        in_specs=[_row_spec(F_IN), _part_spec(F_IN)],
        out_specs=[_row_spec(1), _row_spec(F_IN)],
        out_shape=[
            jax.ShapeDtypeStruct((N_PAD, 1), jnp.float32),
            jax.ShapeDtypeStruct((N_PAD, F_IN), jnp.float32),
        ],
    )(x, degp)


def _tc_l1_body(y1_ref, xs1_ref, dinv_ref, w1_ref, b1_ref, w2_ref,
                a_ref, b_ref):
    dinv = dinv_ref[...]
    z1 = dinv * (y1_ref[0, 0] + y1_ref[0, 1] + xs1_ref[...])
    h1 = jnp.maximum(
        jnp.dot(z1, w1_ref[...], preferred_element_type=jnp.float32)
        + b1_ref[...], 0.0)
    g2 = jnp.dot(h1, w2_ref[...], preferred_element_type=jnp.float32)
    xs2 = dinv * g2
    a_ref[...] = xs2[:, :F_IN]
    b_ref[...] = xs2[:, F_IN:]


def _tc_l1(y1, xs1, dinv, W1, b1, W2):
    return pl.pallas_call(
        _tc_l1_body,
        grid=(_GRID,),
        in_specs=[_part1_spec(F_IN), _row_spec(F_IN), _row_spec(1),
                  _full_spec(F_IN, HID), _full_spec(1, HID),
                  _full_spec(HID, HID)],
        out_specs=[_row_spec(F_IN), _row_spec(F_IN)],
        out_shape=[
            jax.ShapeDtypeStruct((N_PAD, F_IN), jnp.float32),
            jax.ShapeDtypeStruct((N_PAD, F_IN), jnp.float32),
        ],
    )(y1, xs1, dinv, W1, b1, W2)


def _tc_l2_body(y2_ref, xs2a_ref, xs2b_ref, dinv_ref, b2_ref,
                w3_ref, xs3_ref):
    dinv = dinv_ref[...]
    z2a = (dinv * (y2_ref[0, 0] + y2_ref[0, 1] + xs2a_ref[...])
           + b2_ref[:, :F_IN])
    z2b = (dinv * (y2_ref[1, 0] + y2_ref[1, 1] + xs2b_ref[...])
           + b2_ref[:, F_IN:])
    h2 = jnp.maximum(jnp.concatenate([z2a, z2b], axis=1), 0.0)
    g3 = jnp.dot(h2, w3_ref[...], preferred_element_type=jnp.float32)
    xs3_ref[...] = dinv * g3


def _tc_l2(y2, xs2a, xs2b, dinv, b2, W3):
    return pl.pallas_call(
        _tc_l2_body,
        grid=(_GRID,),
        in_specs=[pl.BlockSpec((2, NC, _BLK, F_IN), lambda i: (0, 0, i, 0)),
                  _row_spec(F_IN), _row_spec(F_IN), _row_spec(1),
                  _full_spec(1, HID), _full_spec(HID, F_IN)],
        out_specs=[_row_spec(F_IN)],
        out_shape=[jax.ShapeDtypeStruct((N_PAD, F_IN), jnp.float32)],
    )(y2, xs2a, xs2b, dinv, b2, W3)[0]


def _tc_fin_body(y3_ref, xs3_ref, dinv_ref, b3_ref, out_ref):
    dinv = dinv_ref[...]
    out_ref[...] = (dinv * (y3_ref[0, 0] + y3_ref[0, 1] + xs3_ref[...])
                    + b3_ref[...])


def _tc_fin(y3, xs3, dinv, b3):
    return pl.pallas_call(
        _tc_fin_body,
        grid=(_GRID,),
        in_specs=[_part1_spec(F_IN), _row_spec(F_IN), _row_spec(1),
                  _full_spec(1, F_IN)],
        out_specs=[_row_spec(F_IN)],
        out_shape=[jax.ShapeDtypeStruct((N_PAD, F_IN), jnp.float32)],
    )(y3, xs3, dinv, b3)[0]


def kernel(x, edge_index, W1, b1, W2, b2, W3, b3):
    E = edge_index.shape[1]
    per_w = CK * NW
    groups = -(-E // (per_w * GP))
    ep = groups * GP * per_w

    src = edge_index[0]
    dst = edge_index[1]
    pad = ep - E
    src_slab = jnp.concatenate(
        [src, jnp.zeros((pad,), jnp.int32)]).reshape(NW, groups, GP, CK)
    dummy = N + (jnp.arange(pad, dtype=jnp.int32) % (N_PAD - N))
    dst_slab = jnp.concatenate(
        [dst, dummy]).reshape(NW, groups, GP, CK)

    zeros128 = jnp.zeros((N_PAD, F_IN), jnp.float32)
    ones128 = jnp.ones((CK, F_IN), jnp.float32)
    xp = jnp.concatenate([x, jnp.zeros((N_PAD - N, F_IN), jnp.float32)])

    sc_deg = _make_sc_deg(groups)
    sc_agg1 = _make_sc_agg(groups, 1)
    sc_agg2 = _make_sc_agg(groups, 2)

    degp = sc_deg(dst_slab, ones128, zeros128)
    dinv, xs1 = _tc_prep(xp, degp)

    y1 = sc_agg1(xs1, src_slab, dst_slab, zeros128)
    xs2a, xs2b = _tc_l1(y1, xs1, dinv, W1, b1.reshape(1, HID), W2)

    y2 = sc_agg2(xs2a, xs2b, src_slab, dst_slab, zeros128)
    xs3 = _tc_l2(y2, xs2a, xs2b, dinv, b2.reshape(1, HID), W3)

    y3 = sc_agg1(xs3, src_slab, dst_slab, zeros128)
    return _tc_fin(y3, xs3, dinv, b3.reshape(1, F_IN))[:N]


# speedup vs baseline: 24.3287x; 3.0277x over previous
import functools

import jax
import jax.numpy as jnp
from jax import lax
from jax.experimental import pallas as pl
from jax.experimental.pallas import tpu as pltpu
from jax.experimental.pallas import tpu_sc as plsc

N = 10000
F_IN = 128
HID = 256
NC = 2
NS = 16
NW = NC * NS
CK = 128
GP = 8
N_PAD = 10112
ROWS_T = N_PAD // NS

_mesh = functools.partial(
    plsc.VectorSubcoreMesh,
    core_axis_name="c", subcore_axis_name="s", num_cores=NC, num_subcores=NS,
)


def _wid():
    return lax.axis_index("c") * NS + lax.axis_index("s")


def _make_sc_deg(groups):
    @functools.partial(
        pl.kernel,
        out_type=jax.ShapeDtypeStruct((NC, N_PAD, F_IN), jnp.float32),
        mesh=_mesh(),
        scratch_types=[
            pltpu.VMEM((groups, GP, CK), jnp.int32),
            pltpu.VMEM((CK, F_IN), jnp.float32),
            pltpu.VMEM_SHARED((N_PAD, F_IN), jnp.float32),
        ],
    )
    def sc_deg(dst_hbm, ones_hbm, zeros_hbm, out_hbm, dst_v, ones_v, acc):
        c = lax.axis_index("c")
        s = lax.axis_index("s")
        w = _wid()
        pltpu.sync_copy(zeros_hbm.at[pl.ds(s * ROWS_T, ROWS_T)],
                        acc.at[pl.ds(s * ROWS_T, ROWS_T)])
        pltpu.sync_copy(dst_hbm.at[w], dst_v)
        pltpu.sync_copy(ones_hbm, ones_v)
        plsc.subcore_barrier()

        def body(g, carry):
            for j in range(GP):
                pltpu.sync_copy(ones_v, acc.at[dst_v.at[g, j]], add=True)
            return carry

        lax.fori_loop(0, groups, body, 0)
        plsc.subcore_barrier()
        pltpu.sync_copy(acc.at[pl.ds(s * ROWS_T, ROWS_T)],
                        out_hbm.at[c, pl.ds(s * ROWS_T, ROWS_T)])

    return sc_deg


def _make_sc_agg(groups, ntab):
    @functools.partial(
        pl.kernel,
        out_type=jax.ShapeDtypeStruct((ntab, NC, N_PAD, F_IN), jnp.float32),
        mesh=_mesh(),
        scratch_types=[
            pltpu.VMEM((2, GP, CK), jnp.int32),
            pltpu.VMEM((2, GP, CK), jnp.int32),
            pltpu.VMEM((2, CK, F_IN), jnp.float32),
            pltpu.VMEM_SHARED((N_PAD, F_IN), jnp.float32),
            pltpu.SemaphoreType.DMA,
            pltpu.SemaphoreType.DMA,
            pltpu.SemaphoreType.DMA,
            pltpu.SemaphoreType.DMA,
        ],
    )
    def sc_agg(*refs):
        tables = refs[:ntab]
        src_hbm, dst_hbm, zeros_hbm, out_hbm = refs[ntab:ntab + 4]
        (src_v, dst_v, rows, acc, sem0, sem1, semis, semid) = refs[ntab + 4:]
        c = lax.axis_index("c")
        s = lax.axis_index("s")
        w = _wid()
        gsems = (sem0, sem1)

        for t in range(ntab):
            table_hbm = tables[t]
            pltpu.sync_copy(zeros_hbm.at[pl.ds(s * ROWS_T, ROWS_T)],
                            acc.at[pl.ds(s * ROWS_T, ROWS_T)])
            pltpu.sync_copy(src_hbm.at[w, 0], src_v.at[0])
            pltpu.sync_copy(dst_hbm.at[w, 0], dst_v.at[0])
            plsc.subcore_barrier()

            def gather(buf, idx, sem):
                return pltpu.async_copy(table_hbm.at[idx], rows.at[buf], sem)

            def gwait(buf, idx, sem):
                pltpu.make_async_copy(table_hbm.at[idx], rows.at[buf],
                                      sem).wait()

            gather(0, src_v.at[0, 0], sem0)

            def body(g, carry):
                cur = g % 2
                nxt = (g + 1) % 2

                @pl.when(g < groups - 1)
                def _prefetch():
                    pltpu.async_copy(src_hbm.at[w, g + 1], src_v.at[nxt],
                                     semis)
                    pltpu.async_copy(dst_hbm.at[w, g + 1], dst_v.at[nxt],
                                     semid)

                for j in range(GP):
                    p = j % 2
                    if j < GP - 1:
                        gather(1 - p, src_v.at[cur, j + 1], gsems[1 - p])
                    else:
                        @pl.when(g < groups - 1)
                        def _next_group():
                            pltpu.make_async_copy(src_hbm.at[w, g + 1],
                                                  src_v.at[nxt], semis).wait()
                            pltpu.make_async_copy(dst_hbm.at[w, g + 1],
                                                  dst_v.at[nxt], semid).wait()
                            gather(0, src_v.at[nxt, 0], gsems[0])
                    gwait(p, src_v.at[cur, j], gsems[p])
                    pltpu.sync_copy(rows.at[p], acc.at[dst_v.at[cur, j]],
                                    add=True)
                return carry

            lax.fori_loop(0, groups, body, 0)
            plsc.subcore_barrier()
            pltpu.sync_copy(acc.at[pl.ds(s * ROWS_T, ROWS_T)],
                            out_hbm.at[t, c, pl.ds(s * ROWS_T, ROWS_T)])

    return sc_agg


_BLK = 1264
_GRID = N_PAD // _BLK


def _row_spec(w):
    return pl.BlockSpec((_BLK, w), lambda i: (i, 0))


def _part_spec(w):
    return pl.BlockSpec((NC, _BLK, w), lambda i: (0, i, 0))


def _part1_spec(w):
    return pl.BlockSpec((1, NC, _BLK, w), lambda i: (0, 0, i, 0))


def _full_spec(h, w):
    return pl.BlockSpec((h, w), lambda i: (0, 0))


def _tc_prep_body(x_ref, degp_ref, dinv_ref, xs_ref):
    deg = 1.0 + degp_ref[0, :, 0:1] + degp_ref[1, :, 0:1]
    dinv = lax.rsqrt(deg)
    dinv_ref[...] = dinv
    xs_ref[...] = x_ref[...] * dinv


def _tc_prep(x, degp):
    return pl.pallas_call(
        _tc_prep_body,
        grid=(_GRID,),
        in_specs=[_row_spec(F_IN), _part_spec(F_IN)],
        out_specs=[_row_spec(1), _row_spec(F_IN)],
        out_shape=[
            jax.ShapeDtypeStruct((N_PAD, 1), jnp.float32),
            jax.ShapeDtypeStruct((N_PAD, F_IN), jnp.float32),
        ],
    )(x, degp)


def _tc_l1_body(y1_ref, xs1_ref, dinv_ref, w1_ref, b1_ref, w2_ref,
                a_ref, b_ref):
    dinv = dinv_ref[...]
    z1 = dinv * (y1_ref[0, 0] + y1_ref[0, 1] + xs1_ref[...])
    h1 = jnp.maximum(
        jnp.dot(z1, w1_ref[...], preferred_element_type=jnp.float32)
        + b1_ref[...], 0.0)
    g2 = jnp.dot(h1, w2_ref[...], preferred_element_type=jnp.float32)
    xs2 = dinv * g2
    a_ref[...] = xs2[:, :F_IN]
    b_ref[...] = xs2[:, F_IN:]


def _tc_l1(y1, xs1, dinv, W1, b1, W2):
    return pl.pallas_call(
        _tc_l1_body,
        grid=(_GRID,),
        in_specs=[_part1_spec(F_IN), _row_spec(F_IN), _row_spec(1),
                  _full_spec(F_IN, HID), _full_spec(1, HID),
                  _full_spec(HID, HID)],
        out_specs=[_row_spec(F_IN), _row_spec(F_IN)],
        out_shape=[
            jax.ShapeDtypeStruct((N_PAD, F_IN), jnp.float32),
            jax.ShapeDtypeStruct((N_PAD, F_IN), jnp.float32),
        ],
    )(y1, xs1, dinv, W1, b1, W2)


def _tc_l2_body(y2_ref, xs2a_ref, xs2b_ref, dinv_ref, b2_ref,
                w3_ref, xs3_ref):
    dinv = dinv_ref[...]
    z2a = (dinv * (y2_ref[0, 0] + y2_ref[0, 1] + xs2a_ref[...])
           + b2_ref[:, :F_IN])
    z2b = (dinv * (y2_ref[1, 0] + y2_ref[1, 1] + xs2b_ref[...])
           + b2_ref[:, F_IN:])
    h2 = jnp.maximum(jnp.concatenate([z2a, z2b], axis=1), 0.0)
    g3 = jnp.dot(h2, w3_ref[...], preferred_element_type=jnp.float32)
    xs3_ref[...] = dinv * g3


def _tc_l2(y2, xs2a, xs2b, dinv, b2, W3):
    return pl.pallas_call(
        _tc_l2_body,
        grid=(_GRID,),
        in_specs=[pl.BlockSpec((2, NC, _BLK, F_IN), lambda i: (0, 0, i, 0)),
                  _row_spec(F_IN), _row_spec(F_IN), _row_spec(1),
                  _full_spec(1, HID), _full_spec(HID, F_IN)],
        out_specs=[_row_spec(F_IN)],
        out_shape=[jax.ShapeDtypeStruct((N_PAD, F_IN), jnp.float32)],
    )(y2, xs2a, xs2b, dinv, b2, W3)[0]


def _tc_fin_body(y3_ref, xs3_ref, dinv_ref, b3_ref, out_ref):
    dinv = dinv_ref[...]
    out_ref[...] = (dinv * (y3_ref[0, 0] + y3_ref[0, 1] + xs3_ref[...])
                    + b3_ref[...])


def _tc_fin(y3, xs3, dinv, b3):
    return pl.pallas_call(
        _tc_fin_body,
        grid=(_GRID,),
        in_specs=[_part1_spec(F_IN), _row_spec(F_IN), _row_spec(1),
                  _full_spec(1, F_IN)],
        out_specs=[_row_spec(F_IN)],
        out_shape=[jax.ShapeDtypeStruct((N_PAD, F_IN), jnp.float32)],
    )(y3, xs3, dinv, b3)[0]


def kernel(x, edge_index, W1, b1, W2, b2, W3, b3):
    E = edge_index.shape[1]
    per_w = CK * NW
    groups = -(-E // (per_w * GP))
    ep = groups * GP * per_w

    src = edge_index[0]
    dst = edge_index[1]
    pad = ep - E
    dummy_s = jnp.arange(pad, dtype=jnp.int32) % N
    dummy_d = N + (jnp.arange(pad, dtype=jnp.int32) % (N_PAD - N))
    src_slab = jnp.concatenate(
        [src, dummy_s]).reshape(NW, groups, GP, CK)
    dst_slab = jnp.concatenate(
        [dst, dummy_d]).reshape(NW, groups, GP, CK)

    zeros128 = jnp.zeros((N_PAD, F_IN), jnp.float32)
    ones128 = jnp.ones((CK, F_IN), jnp.float32)
    xp = jnp.concatenate([x, jnp.zeros((N_PAD - N, F_IN), jnp.float32)])

    sc_deg = _make_sc_deg(groups)
    sc_agg1 = _make_sc_agg(groups, 1)
    sc_agg2 = _make_sc_agg(groups, 2)

    degp = sc_deg(dst_slab, ones128, zeros128)
    dinv, xs1 = _tc_prep(xp, degp)

    y1 = sc_agg1(xs1, src_slab, dst_slab, zeros128)
    xs2a, xs2b = _tc_l1(y1, xs1, dinv, W1, b1.reshape(1, HID), W2)

    y2 = sc_agg2(xs2a, xs2b, src_slab, dst_slab, zeros128)
    xs3 = _tc_l2(y2, xs2a, xs2b, dinv, b2.reshape(1, HID), W3)

    y3 = sc_agg1(xs3, src_slab, dst_slab, zeros128)
    return _tc_fin(y3, xs3, dinv, b3.reshape(1, F_IN))[:N]
